# baseline (speedup 1.0000x reference)
import functools

import jax
import jax.numpy as jnp
from jax import lax
from jax.experimental import pallas as pl
from jax.experimental.pallas import tpu as pltpu
from jax.experimental.pallas import tpu_sc as plsc


def _gather_call(x_flat, w_flat, num_feat, rows_per_w, chunk):
    n_chunks = rows_per_w // chunk
    total_rows = x_flat.shape[0]
    d = w_flat.shape[1]
    vocab = w_flat.shape[0] // num_feat

    mesh = plsc.VectorSubcoreMesh(core_axis_name="c", subcore_axis_name="s")

    @functools.partial(
        pl.kernel,
        mesh=mesh,
        compiler_params=pltpu.CompilerParams(use_tc_tiling_on_sc=False),
        out_type=jax.ShapeDtypeStruct((total_rows, d), jnp.float32),
        scratch_types=(
            [pltpu.SMEM((chunk,), jnp.int32) for _ in range(2)]
            + [pltpu.VMEM((chunk,), jnp.int32) for _ in range(2)]
            + [pltpu.VMEM((chunk, d), jnp.float32) for _ in range(2)]
            + [pltpu.SemaphoreType.DMA for _ in range(6)]
        ),
    )
    def k(x_hbm, w_hbm, out_hbm, idx0, idx1, idxv0, idxv1, rows0, rows1,
          semi0, semi1, semg0, semg1, semo0, semo1):
        idx_b = (idx0, idx1)
        idxv_b = (idxv0, idxv1)
        rows_b = (rows0, rows1)
        semi = (semi0, semi1)
        semg = (semg0, semg1)
        semo = (semo0, semo1)

        wid = lax.axis_index("s") * 2 + lax.axis_index("c")
        wbase = wid * rows_per_w

        def row_slice(c):
            return pl.ds(wbase + c * chunk, chunk)

        def enqueue_gathers(b):
            idx_s, rows_v = idx_b[b], rows_b[b]

            def body(j, f):
                t = idx_s[j] + f * vocab
                pltpu.async_copy(
                    w_hbm.at[pl.ds(t, 1)], rows_v.at[pl.ds(j, 1)], semg[b])
                return lax.select(f + 1 == num_feat, 0, f + 1)

            lax.fori_loop(0, chunk, body, 0)

        def drain_gathers(b):
            pltpu.make_async_copy(
                w_hbm.at[pl.ds(0, chunk)], rows_b[b], semg[b]).wait()

        idx_d = [None] * n_chunks
        out_d = [None] * n_chunks
        idx_d[0] = pltpu.async_copy(x_hbm.at[row_slice(0)], idxv_b[0], semi[0])
        for c in range(n_chunks):
            b = c % 2
            idx_d[c].wait()

            def spill(g, _):
                v = idxv_b[b][pl.ds(g * 16, 16)]
                for i in range(16):
                    idx_b[b][g * 16 + i] = v[i]
                return 0

            lax.fori_loop(0, chunk // 16, spill, 0)
            if c + 1 < n_chunks:
                nb = (c + 1) % 2
                idx_d[c + 1] = pltpu.async_copy(
                    x_hbm.at[row_slice(c + 1)], idxv_b[nb], semi[nb])
            if c >= 2:
                out_d[c - 2].wait()
            enqueue_gathers(b)
            if c >= 1:
                drain_gathers(1 - b)
                out_d[c - 1] = pltpu.async_copy(
                    rows_b[1 - b], out_hbm.at[row_slice(c - 1)], semo[1 - b])
        last_b = (n_chunks - 1) % 2
        drain_gathers(last_b)
        out_d[n_chunks - 1] = pltpu.async_copy(
            rows_b[last_b], out_hbm.at[row_slice(n_chunks - 1)], semo[last_b])
        out_d[n_chunks - 2].wait()
        out_d[n_chunks - 1].wait()

    return k(x_flat, w_flat)


def kernel(x, W):
    num_feat, vocab, d = W.shape
    batch = x.shape[0]
    total_rows = batch * num_feat

    nw = 32
    rows_per_w = total_rows // nw
    chunk = 832

    x_flat = x.reshape(total_rows).astype(jnp.int32)
    w_flat = W.reshape(num_feat * vocab, d)
    out = _gather_call(x_flat, w_flat, num_feat, rows_per_w, chunk)
    return out.reshape(batch, num_feat, d)
